# baseline (speedup 1.0000x reference)
import functools

import jax
import jax.numpy as jnp
from jax import lax
from jax.experimental import pallas as pl
from jax.experimental.pallas import tpu as pltpu
from jax.experimental.pallas import tpu_sc as plsc

_CHUNK = 128
_NBUF = 5


@functools.lru_cache(maxsize=None)
def _make_gather(V, D, B, NC, NS):
    NW = NC * NS
    assert B % (NW * _CHUNK) == 0
    b_per_w = B // NW
    n_chunks = b_per_w // _CHUNK
    assert n_chunks % _NBUF == 0
    n_groups = n_chunks // _NBUF
    mesh = plsc.VectorSubcoreMesh(core_axis_name="c", subcore_axis_name="s")

    @functools.partial(
        pl.kernel,
        mesh=mesh,
        out_type=jax.ShapeDtypeStruct((B, D), jnp.float32),
        scratch_types=(
            [pltpu.VMEM((n_chunks, _CHUNK), jnp.int32)]
            + [pltpu.VMEM((_CHUNK, D), jnp.float32) for _ in range(_NBUF)]
            + [pltpu.SemaphoreType.DMA for _ in range(2 * _NBUF)]
        ),
    )
    def gather_kernel(table_hbm, idx_hbm, out_hbm, idx_v, *rest):
        bufs = rest[:_NBUF]
        gsems = rest[_NBUF : 2 * _NBUF]
        ssems = rest[2 * _NBUF :]
        wid = lax.axis_index("s") * NC + lax.axis_index("c")
        row_base = wid * b_per_w
        pltpu.sync_copy(idx_hbm.at[wid], idx_v)

        def start_gather(c, b):
            pltpu.async_copy(table_hbm.at[idx_v.at[c]], bufs[b], gsems[b])

        def wait_gather(c, b):
            pltpu.make_async_copy(table_hbm.at[idx_v.at[c]], bufs[b], gsems[b]).wait()

        def out_slice(c):
            return out_hbm.at[pl.ds(row_base + c * _CHUNK, _CHUNK)]

        def start_scatter(c, b):
            pltpu.async_copy(bufs[b], out_slice(c), ssems[b])

        def wait_scatter(c, b):
            pltpu.make_async_copy(bufs[b], out_slice(c), ssems[b]).wait()

        for b in range(_NBUF):
            start_gather(b, b)

        def group(g, carry):
            c0 = g * _NBUF
            for b in range(_NBUF):
                wait_gather(c0 + b, b)
                start_scatter(c0 + b, b)
            for b in range(_NBUF):
                wait_scatter(c0 + b, b)
                start_gather(c0 + _NBUF + b, b)
            return carry

        lax.fori_loop(0, n_groups - 1, group, 0)

        c0 = (n_groups - 1) * _NBUF
        for b in range(_NBUF):
            wait_gather(c0 + b, b)
            start_scatter(c0 + b, b)
        for b in range(_NBUF):
            wait_scatter(c0 + b, b)

    return gather_kernel


def kernel(token_ids, embeddings):
    Bt, S = token_ids.shape
    V, D = embeddings.shape
    B = Bt * S
    info = plsc.get_sparse_core_info()
    NC, NS = info.num_cores, info.num_subcores
    NW = NC * NS
    idx = token_ids.astype(jnp.int32).reshape(NW, B // (NW * _CHUNK), _CHUNK)
    out = _make_gather(V, D, B, NC, NS)(embeddings, idx)
    return out.reshape(Bt, S, D)


# speedup vs baseline: 1.0158x; 1.0158x over previous
import functools

import jax
import jax.numpy as jnp
from jax import lax
from jax.experimental import pallas as pl
from jax.experimental.pallas import tpu as pltpu
from jax.experimental.pallas import tpu_sc as plsc

_CHUNK = 64
_NBUF = 10


@functools.lru_cache(maxsize=None)
def _make_gather(V, D, B, NC, NS):
    NW = NC * NS
    assert B % (NW * _CHUNK) == 0
    b_per_w = B // NW
    n_chunks = b_per_w // _CHUNK
    assert n_chunks % _NBUF == 0
    n_groups = n_chunks // _NBUF
    mesh = plsc.VectorSubcoreMesh(core_axis_name="c", subcore_axis_name="s")

    @functools.partial(
        pl.kernel,
        mesh=mesh,
        out_type=jax.ShapeDtypeStruct((B, D), jnp.float32),
        scratch_types=(
            [pltpu.VMEM((n_chunks, _CHUNK), jnp.int32)]
            + [pltpu.VMEM((_CHUNK, D), jnp.float32) for _ in range(_NBUF)]
            + [pltpu.SemaphoreType.DMA for _ in range(2 * _NBUF)]
        ),
    )
    def gather_kernel(table_hbm, idx_hbm, out_hbm, idx_v, *rest):
        bufs = rest[:_NBUF]
        gsems = rest[_NBUF : 2 * _NBUF]
        ssems = rest[2 * _NBUF :]
        wid = lax.axis_index("s") * NC + lax.axis_index("c")
        row_base = wid * b_per_w
        pltpu.sync_copy(idx_hbm.at[wid], idx_v)

        def start_gather(c, b):
            pltpu.async_copy(table_hbm.at[idx_v.at[c]], bufs[b], gsems[b])

        def wait_gather(c, b):
            pltpu.make_async_copy(table_hbm.at[idx_v.at[c]], bufs[b], gsems[b]).wait()

        def out_slice(c):
            return out_hbm.at[pl.ds(row_base + c * _CHUNK, _CHUNK)]

        def start_scatter(c, b):
            pltpu.async_copy(bufs[b], out_slice(c), ssems[b])

        def wait_scatter(c, b):
            pltpu.make_async_copy(bufs[b], out_slice(c), ssems[b]).wait()

        for b in range(_NBUF):
            start_gather(b, b)

        def group(g, carry):
            c0 = g * _NBUF
            for b in range(_NBUF):
                wait_gather(c0 + b, b)
                start_scatter(c0 + b, b)
            for b in range(_NBUF):
                wait_scatter(c0 + b, b)
                start_gather(c0 + _NBUF + b, b)
            return carry

        lax.fori_loop(0, n_groups - 1, group, 0)

        c0 = (n_groups - 1) * _NBUF
        for b in range(_NBUF):
            wait_gather(c0 + b, b)
            start_scatter(c0 + b, b)
        for b in range(_NBUF):
            wait_scatter(c0 + b, b)

    return gather_kernel


def kernel(token_ids, embeddings):
    Bt, S = token_ids.shape
    V, D = embeddings.shape
    B = Bt * S
    info = plsc.get_sparse_core_info()
    NC, NS = info.num_cores, info.num_subcores
    NW = NC * NS
    idx = token_ids.astype(jnp.int32).reshape(NW, B // (NW * _CHUNK), _CHUNK)
    out = _make_gather(V, D, B, NC, NS)(embeddings, idx)
    return out.reshape(Bt, S, D)


# speedup vs baseline: 1.0271x; 1.0112x over previous
import functools

import jax
import jax.numpy as jnp
from jax import lax
from jax.experimental import pallas as pl
from jax.experimental.pallas import tpu as pltpu
from jax.experimental.pallas import tpu_sc as plsc

_CHUNK = 64
_G = 5
_NBLK = 2


@functools.lru_cache(maxsize=None)
def _make_gather(V, D, B, NC, NS):
    NW = NC * NS
    assert B % (NW * _CHUNK) == 0
    b_per_w = B // NW
    n_chunks = b_per_w // _CHUNK
    blk_rows = _G * _CHUNK
    assert n_chunks % _G == 0
    n_blocks = n_chunks // _G
    assert n_blocks % _NBLK == 0
    n_groups = n_blocks // _NBLK
    mesh = plsc.VectorSubcoreMesh(core_axis_name="c", subcore_axis_name="s")

    @functools.partial(
        pl.kernel,
        mesh=mesh,
        out_type=jax.ShapeDtypeStruct((B, D), jnp.float32),
        scratch_types=(
            [pltpu.VMEM((n_chunks, _CHUNK), jnp.int32)]
            + [pltpu.VMEM((blk_rows, D), jnp.float32) for _ in range(_NBLK)]
            + [pltpu.SemaphoreType.DMA for _ in range(_NBLK * _G)]
            + [pltpu.SemaphoreType.DMA for _ in range(_NBLK)]
        ),
    )
    def gather_kernel(table_hbm, idx_hbm, out_hbm, idx_v, *rest):
        bufs = rest[:_NBLK]
        gsems = rest[_NBLK : _NBLK + _NBLK * _G]
        ssems = rest[_NBLK + _NBLK * _G :]
        wid = lax.axis_index("s") * NC + lax.axis_index("c")
        row_base = wid * b_per_w
        pltpu.sync_copy(idx_hbm.at[wid], idx_v)

        def gather_args(blk, b, j):
            return (
                table_hbm.at[idx_v.at[blk * _G + j]],
                bufs[b].at[pl.ds(j * _CHUNK, _CHUNK)],
                gsems[b * _G + j],
            )

        def start_gathers(blk, b):
            for j in range(_G):
                pltpu.async_copy(*gather_args(blk, b, j))

        def wait_gathers(blk, b):
            for j in range(_G):
                pltpu.make_async_copy(*gather_args(blk, b, j)).wait()

        def out_slice(blk):
            return out_hbm.at[pl.ds(row_base + blk * blk_rows, blk_rows)]

        def start_scatter(blk, b):
            pltpu.async_copy(bufs[b], out_slice(blk), ssems[b])

        def wait_scatter(blk, b):
            pltpu.make_async_copy(bufs[b], out_slice(blk), ssems[b]).wait()

        for b in range(_NBLK):
            start_gathers(b, b)

        def group(g, carry):
            blk0 = g * _NBLK
            for b in range(_NBLK):
                wait_gathers(blk0 + b, b)
                start_scatter(blk0 + b, b)
            for b in range(_NBLK):
                wait_scatter(blk0 + b, b)
                start_gathers(blk0 + _NBLK + b, b)
            return carry

        lax.fori_loop(0, n_groups - 1, group, 0)

        blk0 = (n_groups - 1) * _NBLK
        for b in range(_NBLK):
            wait_gathers(blk0 + b, b)
            start_scatter(blk0 + b, b)
        for b in range(_NBLK):
            wait_scatter(blk0 + b, b)

    return gather_kernel


def kernel(token_ids, embeddings):
    Bt, S = token_ids.shape
    V, D = embeddings.shape
    B = Bt * S
    info = plsc.get_sparse_core_info()
    NC, NS = info.num_cores, info.num_subcores
    NW = NC * NS
    idx = token_ids.astype(jnp.int32).reshape(NW, B // (NW * _CHUNK), _CHUNK)
    out = _make_gather(V, D, B, NC, NS)(embeddings, idx)
    return out.reshape(Bt, S, D)
